# baseline (speedup 1.0000x reference)
import functools

import jax
import jax.numpy as jnp
from jax.experimental import pallas as pl
from jax.experimental.pallas import tpu as pltpu

TR_ = 0.5
UNIT_ = 16
N_, C_, H_, W_ = 8, 16, 384, 384
NC_ = N_ * C_
NEG_ = float("-inf")


def _minmax_kernel(x_ref, mn_ref, mx_ref):
    i = pl.program_id(0)
    z = x_ref[...]
    zmin = jnp.min(z)
    zmax = jnp.max(z)

    @pl.when(i == 0)
    def _():
        mn_ref[0, 0] = zmin
        mx_ref[0, 0] = zmax

    @pl.when(i != 0)
    def _():
        mn_ref[0, 0] = jnp.minimum(mn_ref[0, 0], zmin)
        mx_ref[0, 0] = jnp.maximum(mx_ref[0, 0], zmax)


def _roll_up(x, s, axis):
    dim = x.shape[axis]
    return pltpu.roll(x, (dim - s) % dim, axis)


def _main_kernel(th_ref, x_ref, crops_ref, fb_ref):
    H, W = H_, W_
    r_io = jax.lax.broadcasted_iota(jnp.int32, (H, W), 0)
    c_io = jax.lax.broadcasted_iota(jnp.int32, (H, W), 1)
    diag = c_io - r_io
    for j in range(x_ref.shape[0]):
        _one_map(th_ref, x_ref, crops_ref, fb_ref, diag, j)


def _one_map(th_ref, x_ref, crops_ref, fb_ref, diag, j):
    H, W = H_, W_
    z = x_ref[j]
    thresh = th_ref[0]

    c1 = jax.lax.broadcasted_iota(jnp.int32, (1, W), 1)
    r1 = jax.lax.broadcasted_iota(jnp.int32, (H, 1), 0)

    bx = z >= thresh
    colpres = jnp.max(z, axis=0, keepdims=True) >= thresh
    rowpres = jnp.max(z, axis=1, keepdims=True) >= thresh
    xv = jnp.where(colpres, c1, 0)
    yv = jnp.where(rowpres, r1, 0)

    def last_set(v, io, n):
        m = jnp.max(v)
        return jnp.min(jnp.where(v == m, io, n))

    def first_set(v, io, n):
        big = jnp.max(v) + 1
        w = jnp.where(v == 0, big, v)
        mn = jnp.min(w)
        return jnp.min(jnp.where(w == mn, io, n))

    x_max = last_set(xv, c1, W)
    x_min = first_set(xv, c1, W)
    y_max = last_set(yv, r1, H)
    y_min = first_set(yv, r1, H)

    wh_x = x_max - x_min
    wh_y = y_max - y_min
    uw = jnp.maximum((wh_x + UNIT_ - 1) // UNIT_, 1) * UNIT_
    uh = jnp.maximum((wh_y + UNIT_ - 1) // UNIT_, 1) * UNIT_
    ex = jnp.maximum(uw - wh_x, 0)
    ey = jnp.maximum(uh - wh_y, 0)
    b0 = jnp.maximum(x_min - ex, 0)
    b1 = jnp.maximum(y_min - ey, 0)
    b2 = x_max + ex
    b3 = y_max + ey
    b2c = jnp.minimum(b2, W)
    b3c = jnp.minimum(b3, H)
    hh = b3c - b1
    ww = b2c - b0
    kh = jnp.minimum(hh, uh)
    kw = jnp.minimum(ww, uw)

    xm = jnp.where(bx, 1.0, z)

    rowwin = (c1 >= b1) & (c1 < b3c)
    colwin = (r1 >= b0) & (r1 < b2c)
    mrow = ((diag >= 0) & (diag < kh) & rowwin).astype(jnp.float32)
    mcol = ((diag <= 0) & (diag > -kw) & colwin).astype(jnp.float32)
    dot = functools.partial(
        jnp.dot,
        precision=jax.lax.Precision.DEFAULT,
        preferred_element_type=jnp.float32,
    )
    s = dot(dot(mrow, xm), mcol)

    pr = jnp.where((r1 >= b1) & (r1 <= b3c - kh), 0.0, NEG_)
    pc = jnp.where((c1 >= b0) & (c1 <= b2c - kw), 0.0, NEG_)
    sm = s + pr + pc

    colmax = jnp.max(sm, axis=0, keepdims=True)
    rowmax = jnp.max(sm, axis=1, keepdims=True)
    m1 = jnp.max(colmax)
    x0 = jnp.min(jnp.where(colmax == m1, c1, W))
    y0 = jnp.min(jnp.where(rowmax == m1, r1, H))

    empty = (hh <= 0) | (ww <= 0)
    x0 = jnp.where(empty, b0, x0)
    y0 = jnp.where(empty, b1, y0)

    rmask = r1 <= jnp.minimum((H - 1) - y0, uh - 1)
    cmask = c1 <= jnp.minimum((W - 1) - x0, uw - 1)
    rcrop = jnp.where(rmask, _roll_up(xm, y0, 0), 0.0)
    ccrop = jnp.where(cmask, _roll_up(rcrop, x0, 1), 0.0)
    crops_ref[j] = ccrop

    l128 = jax.lax.broadcasted_iota(jnp.int32, (1, 128), 1)
    row = jnp.where(
        l128 == 0,
        x0,
        jnp.where(
            l128 == 1,
            y0,
            jnp.where(l128 == 2, x0 + uw, jnp.where(l128 == 3, y0 + uh, 0)),
        ),
    )
    fb_ref[j] = row


@jax.jit
def kernel(X):
    x3 = X.reshape(NC_, H_, W_)
    mn, mx = pl.pallas_call(
        _minmax_kernel,
        grid=(NC_ // 4,),
        in_specs=[pl.BlockSpec((4, H_, W_), lambda i: (i, 0, 0))],
        out_specs=[
            pl.BlockSpec(memory_space=pltpu.SMEM),
            pl.BlockSpec(memory_space=pltpu.SMEM),
        ],
        out_shape=[
            jax.ShapeDtypeStruct((1, 1), jnp.float32),
            jax.ShapeDtypeStruct((1, 1), jnp.float32),
        ],
    )(x3)
    thresh = (mn + (mx - mn) * TR_).reshape(1)

    crops, fb3 = pl.pallas_call(
        _main_kernel,
        grid=(NC_ // 4,),
        in_specs=[
            pl.BlockSpec(memory_space=pltpu.SMEM),
            pl.BlockSpec((4, H_, W_), lambda i: (i, 0, 0)),
        ],
        out_specs=[
            pl.BlockSpec((4, H_, W_), lambda i: (i, 0, 0)),
            pl.BlockSpec((4, 1, 128), lambda i: (i, 0, 0)),
        ],
        out_shape=[
            jax.ShapeDtypeStruct((NC_, H_, W_), jnp.float32),
            jax.ShapeDtypeStruct((NC_, 1, 128), jnp.int32),
        ],
    )(thresh, x3)

    out = crops.reshape(N_, C_, H_, W_)
    fb = fb3[:, 0, :4]
    return out, fb


# speedup vs baseline: 1.0129x; 1.0129x over previous
import functools

import jax
import jax.numpy as jnp
from jax.experimental import pallas as pl
from jax.experimental.pallas import tpu as pltpu

TR_ = 0.5
UNIT_ = 16
N_, C_, H_, W_ = 8, 16, 384, 384
NC_ = N_ * C_
NEG_ = float("-inf")


def _minmax_kernel(x_ref, mn_ref, mx_ref):
    i = pl.program_id(0)
    z = x_ref[...]
    zmin = jnp.min(z)
    zmax = jnp.max(z)

    @pl.when(i == 0)
    def _():
        mn_ref[0, 0] = zmin
        mx_ref[0, 0] = zmax

    @pl.when(i != 0)
    def _():
        mn_ref[0, 0] = jnp.minimum(mn_ref[0, 0], zmin)
        mx_ref[0, 0] = jnp.maximum(mx_ref[0, 0], zmax)


def _roll_up(x, s, axis):
    dim = x.shape[axis]
    return pltpu.roll(x, (dim - s) % dim, axis)


def _main_kernel(th_ref, x_ref, crops_ref, fb_ref):
    H, W = H_, W_
    r_io = jax.lax.broadcasted_iota(jnp.int32, (H, W), 0)
    c_io = jax.lax.broadcasted_iota(jnp.int32, (H, W), 1)
    diag = c_io - r_io
    for j in range(x_ref.shape[0]):
        _one_map(th_ref, x_ref, crops_ref, fb_ref, diag, j)


def _one_map(th_ref, x_ref, crops_ref, fb_ref, diag, j):
    H, W = H_, W_
    z = x_ref[j]
    thresh = th_ref[0]

    c1 = jax.lax.broadcasted_iota(jnp.int32, (1, W), 1)
    r1 = jax.lax.broadcasted_iota(jnp.int32, (H, 1), 0)

    bx = z >= thresh
    bxf = bx.astype(jnp.float32)
    colpres = jnp.sum(bxf, axis=0, keepdims=True) > 0.0
    rowpres = jnp.sum(bxf, axis=1, keepdims=True) > 0.0
    xv = jnp.where(colpres, c1, 0)
    yv = jnp.where(rowpres, r1, 0)

    def last_set(v, io, n):
        m = jnp.max(v)
        return jnp.min(jnp.where(v == m, io, n))

    def first_set(v, io, n):
        big = jnp.max(v) + 1
        w = jnp.where(v == 0, big, v)
        mn = jnp.min(w)
        return jnp.min(jnp.where(w == mn, io, n))

    x_max = last_set(xv, c1, W)
    x_min = first_set(xv, c1, W)
    y_max = last_set(yv, r1, H)
    y_min = first_set(yv, r1, H)

    wh_x = x_max - x_min
    wh_y = y_max - y_min
    uw = jnp.maximum((wh_x + UNIT_ - 1) // UNIT_, 1) * UNIT_
    uh = jnp.maximum((wh_y + UNIT_ - 1) // UNIT_, 1) * UNIT_
    ex = jnp.maximum(uw - wh_x, 0)
    ey = jnp.maximum(uh - wh_y, 0)
    b0 = jnp.maximum(x_min - ex, 0)
    b1 = jnp.maximum(y_min - ey, 0)
    b2 = x_max + ex
    b3 = y_max + ey
    b2c = jnp.minimum(b2, W)
    b3c = jnp.minimum(b3, H)
    hh = b3c - b1
    ww = b2c - b0
    kh = jnp.minimum(hh, uh)
    kw = jnp.minimum(ww, uw)

    xm = jnp.where(bx, 1.0, z)

    rowwin = (c1 >= b1) & (c1 < b3c)
    colwin = (r1 >= b0) & (r1 < b2c)
    mrow = ((diag >= 0) & (diag < kh) & rowwin).astype(jnp.float32)
    mcol = ((diag <= 0) & (diag > -kw) & colwin).astype(jnp.float32)
    dot = functools.partial(
        jnp.dot,
        precision=jax.lax.Precision.DEFAULT,
        preferred_element_type=jnp.float32,
    )
    s = dot(dot(mrow, xm), mcol)

    pr = jnp.where((r1 >= b1) & (r1 <= b3c - kh), 0.0, NEG_)
    pc = jnp.where((c1 >= b0) & (c1 <= b2c - kw), 0.0, NEG_)
    sm = s + pr + pc

    colmax = jnp.max(sm, axis=0, keepdims=True)
    rowmax = jnp.max(sm, axis=1, keepdims=True)
    m1 = jnp.max(colmax)
    x0 = jnp.min(jnp.where(colmax == m1, c1, W))
    y0 = jnp.min(jnp.where(rowmax == m1, r1, H))

    empty = (hh <= 0) | (ww <= 0)
    x0 = jnp.where(empty, b0, x0)
    y0 = jnp.where(empty, b1, y0)

    rmask = r1 <= jnp.minimum((H - 1) - y0, uh - 1)
    cmask = c1 <= jnp.minimum((W - 1) - x0, uw - 1)
    rcrop = jnp.where(rmask, _roll_up(xm, y0, 0), 0.0)
    ccrop = jnp.where(cmask, _roll_up(rcrop, x0, 1), 0.0)
    crops_ref[j] = ccrop

    l128 = jax.lax.broadcasted_iota(jnp.int32, (1, 128), 1)
    row = jnp.where(
        l128 == 0,
        x0,
        jnp.where(
            l128 == 1,
            y0,
            jnp.where(l128 == 2, x0 + uw, jnp.where(l128 == 3, y0 + uh, 0)),
        ),
    )
    fb_ref[j] = row


@jax.jit
def kernel(X):
    x3 = X.reshape(NC_, H_, W_)
    mn, mx = pl.pallas_call(
        _minmax_kernel,
        grid=(NC_ // 4,),
        in_specs=[pl.BlockSpec((4, H_, W_), lambda i: (i, 0, 0))],
        out_specs=[
            pl.BlockSpec(memory_space=pltpu.SMEM),
            pl.BlockSpec(memory_space=pltpu.SMEM),
        ],
        out_shape=[
            jax.ShapeDtypeStruct((1, 1), jnp.float32),
            jax.ShapeDtypeStruct((1, 1), jnp.float32),
        ],
    )(x3)
    thresh = (mn + (mx - mn) * TR_).reshape(1)

    crops, fb3 = pl.pallas_call(
        _main_kernel,
        grid=(NC_ // 4,),
        in_specs=[
            pl.BlockSpec(memory_space=pltpu.SMEM),
            pl.BlockSpec((4, H_, W_), lambda i: (i, 0, 0)),
        ],
        out_specs=[
            pl.BlockSpec((4, H_, W_), lambda i: (i, 0, 0)),
            pl.BlockSpec((4, 1, 128), lambda i: (i, 0, 0)),
        ],
        out_shape=[
            jax.ShapeDtypeStruct((NC_, H_, W_), jnp.float32),
            jax.ShapeDtypeStruct((NC_, 1, 128), jnp.int32),
        ],
    )(thresh, x3)

    out = crops.reshape(N_, C_, H_, W_)
    fb = fb3[:, 0, :4]
    return out, fb
